# baseline (speedup 1.0000x reference)
import functools

import jax
import jax.numpy as jnp
from jax.experimental import pallas as pl
from jax.experimental.pallas import tpu as pltpu


def _gcn_body(x_ref, adj_ref, w_ref, b_ref, out_ref, z_ref, h_ref, *, bm, zc):
    l = pl.program_id(0)
    m = pl.program_id(1)
    n = z_ref.shape[0]

    def _project(src_ref):
        def body(i, carry):
            sl = pl.ds(i * zc, zc)
            z_ref[sl, :] = jax.lax.dot_general(
                src_ref[sl, :], w_ref[0], (((1,), (0,)), ((), ())),
                precision=jax.lax.Precision.HIGHEST,
                preferred_element_type=jnp.float32)
            return carry
        jax.lax.fori_loop(0, n // zc, body, 0)

    @pl.when(jnp.logical_and(m == 0, l == 0))
    def _():
        _project(x_ref)

    @pl.when(jnp.logical_and(m == 0, l > 0))
    def _():
        _project(h_ref)

    h = jax.lax.dot_general(
        adj_ref[...], z_ref[...], (((1,), (0,)), ((), ())),
        preferred_element_type=jnp.float32)
    h = jnp.maximum(h + b_ref[0], 0.0)

    @pl.when(l < 2)
    def _():
        h_ref[pl.ds(m * bm, bm), :] = h

    @pl.when(l == 2)
    def _():
        mx = jnp.max(h, axis=1, keepdims=True)
        s = jnp.sum(jnp.exp(h - mx), axis=1, keepdims=True)
        out_ref[...] = h - mx - jnp.log(s)


def kernel(x, adj, W1, b1, W2, b2, W3, b3):
    n, d = x.shape
    w = jnp.stack([W1, W2, W3])
    b = jnp.stack([b1, b2, b3]).reshape(3, 1, d)

    bm = 200 if n % 200 == 0 else n
    nm = n // bm
    zc = 1000 if n % 1000 == 0 else n

    return pl.pallas_call(
        functools.partial(_gcn_body, bm=bm, zc=zc),
        grid=(3, nm),
        in_specs=[
            pl.BlockSpec((n, d), lambda l, m: (0, 0)),
            pl.BlockSpec((bm, n), lambda l, m: (m, 0)),
            pl.BlockSpec((1, d, d), lambda l, m: (l, 0, 0)),
            pl.BlockSpec((1, 1, d), lambda l, m: (l, 0, 0)),
        ],
        out_specs=pl.BlockSpec(
            (bm, d), lambda l, m: (jnp.where(l == 2, m, 0), 0)),
        out_shape=jax.ShapeDtypeStruct((n, d), jnp.float32),
        scratch_shapes=[
            pltpu.VMEM((n, d), jnp.float32),
            pltpu.VMEM((n, d), jnp.float32),
        ],
        compiler_params=pltpu.CompilerParams(
            dimension_semantics=("arbitrary", "arbitrary"),
            vmem_limit_bytes=100 * 1024 * 1024,
        ),
    )(x, adj, w, b)
